# baseline (speedup 1.0000x reference)
import functools

import jax
import jax.numpy as jnp
from jax import lax
from jax.experimental import pallas as pl
from jax.experimental.pallas import tpu as pltpu
from jax.experimental.pallas import tpu_sc as plsc

_NC = 2
_NS = 16
_NW = _NC * _NS
_L = 16

_B = 4
_R = 8192
_C = 4096
_N = _B * _C
_PER_W = _N // _NW
_CHUNKS = 4
_CHUNK = _PER_W // _CHUNKS


def _gather_body(x_hbm, idx_hbm, out_hbm, idx_v, fidx_v, out_v, sem):
    wid = lax.axis_index("s") * _NC + lax.axis_index("c")
    base = wid * _PER_W
    b = base // _C
    cbase = base % _C

    pltpu.sync_copy(idx_hbm.at[wid], idx_v)

    lane = lax.iota(jnp.int32, _L)
    row_off = b * (_R * _C) + cbase
    for r in range(_CHUNKS):
        for i in range(_CHUNK // _L):
            v = idx_v[r, pl.ds(i * _L, _L)]
            off = row_off + r * _CHUNK + i * _L
            fidx_v[r, pl.ds(i * _L, _L)] = v * _C + off + lane

    copies = []
    for r in range(_CHUNKS):
        copies.append(
            pltpu.async_copy(x_hbm.at[fidx_v.at[r]], out_v.at[r], sem)
        )
    for cp in copies:
        cp.wait()

    pltpu.sync_copy(out_v, out_hbm.at[wid])


@jax.jit
def kernel(x, index):
    x_flat = x.reshape(-1)
    idx = index.reshape(_NW, _CHUNKS, _CHUNK)

    mesh = plsc.VectorSubcoreMesh(core_axis_name="c", subcore_axis_name="s")
    run = functools.partial(
        pl.kernel,
        mesh=mesh,
        out_type=jax.ShapeDtypeStruct((_NW, _CHUNKS, _CHUNK), jnp.float32),
        scratch_types=[
            pltpu.VMEM((_CHUNKS, _CHUNK), jnp.int32),
            pltpu.VMEM((_CHUNKS, _CHUNK), jnp.int32),
            pltpu.VMEM((_CHUNKS, _CHUNK), jnp.float32),
            pltpu.SemaphoreType.DMA,
        ],
    )(_gather_body)
    out = run(x_flat, idx)
    return out.reshape(_B, 1, _C)


# speedup vs baseline: 14.2461x; 14.2461x over previous
import functools

import jax
import jax.numpy as jnp
from jax import lax
from jax.experimental import pallas as pl
from jax.experimental.pallas import tpu as pltpu
from jax.experimental.pallas import tpu_sc as plsc

_NC = 2
_NS = 16
_NW = _NC * _NS
_L = 16

_B = 4
_R = 8192
_C = 4096
_N = _B * _C
_BLK = 128
_NBLK = _N // _BLK
_BPW = _NBLK // _NW


def _gather_body(x_hbm, idx_hbm, out_hbm, idx_v, gidx_v, rows_v, out_v, sem):
    wid = lax.axis_index("s") * _NC + lax.axis_index("c")

    pltpu.sync_copy(idx_hbm.at[wid], idx_v)

    for r in range(_BPW):
        g = wid * _BPW + r
        b = g // (_C // _BLK)
        for i in range(_BLK // _L):
            gidx_v[r, pl.ds(i * _L, _L)] = idx_v[r, pl.ds(i * _L, _L)] + b * _R

    copies = []
    for r in range(_BPW):
        g = wid * _BPW + r
        ct = g % (_C // _BLK)
        copies.append(
            pltpu.async_copy(
                x_hbm.at[gidx_v.at[r], pl.ds(ct * _BLK, _BLK)],
                rows_v.at[r],
                sem,
            )
        )
    for cp in copies:
        cp.wait()

    for r in range(_BPW):
        for i in range(_BLK // _L):
            ids = lax.iota(jnp.int32, _L) + i * _L
            out_v[r, pl.ds(i * _L, _L)] = plsc.load_gather(
                rows_v.at[r], [ids, ids]
            )

    pltpu.sync_copy(out_v, out_hbm.at[wid])


@jax.jit
def kernel(x, index):
    x2 = x.reshape(_B * _R, _C)
    idx = index.reshape(_NW, _BPW, _BLK)

    mesh = plsc.VectorSubcoreMesh(core_axis_name="c", subcore_axis_name="s")
    run = functools.partial(
        pl.kernel,
        mesh=mesh,
        out_type=jax.ShapeDtypeStruct((_NW, _BPW, _BLK), jnp.float32),
        scratch_types=[
            pltpu.VMEM((_BPW, _BLK), jnp.int32),
            pltpu.VMEM((_BPW, _BLK), jnp.int32),
            pltpu.VMEM((_BPW, _BLK, _BLK), jnp.float32),
            pltpu.VMEM((_BPW, _BLK), jnp.float32),
            pltpu.SemaphoreType.DMA,
        ],
        compiler_params=pltpu.CompilerParams(
            use_tc_tiling_on_sc=True, needs_layout_passes=False
        ),
    )(_gather_body)
    out = run(x2, idx)
    return out.reshape(_B, 1, _C)


# speedup vs baseline: 14.2688x; 1.0016x over previous
import functools

import jax
import jax.numpy as jnp
from jax import lax
from jax.experimental import pallas as pl
from jax.experimental.pallas import tpu as pltpu
from jax.experimental.pallas import tpu_sc as plsc

_NC = 2
_NS = 16
_NW = _NC * _NS
_L = 16

_B = 4
_R = 8192
_C = 4096
_PER_W = (_B * _C) // _NW
_BLK = 128
_NBLK = _PER_W // _BLK


def _gather_body(x_hbm, idx_hbm, out_hbm, idx_v, gidx_v, rows_v, out_v, sem):
    wid = lax.axis_index("s") * _NC + lax.axis_index("c")
    e0 = wid * _PER_W
    b = e0 // _C
    cbase = e0 % _C

    pltpu.sync_copy(idx_hbm.at[b, 0, pl.ds(cbase, _PER_W)], idx_v)

    for i in range(_PER_W // _L):
        gidx_v[pl.ds(i * _L, _L)] = idx_v[pl.ds(i * _L, _L)] + b * _R

    copies = []
    for j in range(_NBLK):
        copies.append(
            pltpu.async_copy(
                x_hbm.at[gidx_v.at[pl.ds(j * _BLK, _BLK)],
                         pl.ds(cbase + j * _BLK, _BLK)],
                rows_v.at[j],
                sem,
            )
        )
    for cp in copies:
        cp.wait()

    for j in range(_NBLK):
        for i in range(_BLK // _L):
            ids = lax.iota(jnp.int32, _L) + i * _L
            out_v[pl.ds(j * _BLK + i * _L, _L)] = plsc.load_gather(
                rows_v.at[j], [ids, ids]
            )

    pltpu.sync_copy(out_v, out_hbm.at[b, 0, pl.ds(cbase, _PER_W)])


@jax.jit
def kernel(x, index):
    mesh = plsc.VectorSubcoreMesh(core_axis_name="c", subcore_axis_name="s")
    run = functools.partial(
        pl.kernel,
        mesh=mesh,
        out_type=jax.ShapeDtypeStruct((_B, 1, _C), jnp.float32),
        scratch_types=[
            pltpu.VMEM((_PER_W,), jnp.int32),
            pltpu.VMEM((_PER_W,), jnp.int32),
            pltpu.VMEM((_NBLK, _BLK, _BLK), jnp.float32),
            pltpu.VMEM((_PER_W,), jnp.float32),
            pltpu.SemaphoreType.DMA,
        ],
        compiler_params=pltpu.CompilerParams(
            use_tc_tiling_on_sc=True, needs_layout_passes=False
        ),
    )(_gather_body)
    return run(x.reshape(_B * _R, _C), index)


# speedup vs baseline: 18.6927x; 1.3100x over previous
import functools

import jax
import jax.numpy as jnp
from jax import lax
from jax.experimental import pallas as pl
from jax.experimental.pallas import tpu as pltpu
from jax.experimental.pallas import tpu_sc as plsc

_NC = 2
_NS = 16
_NW = _NC * _NS
_L = 16

_B = 4
_R = 8192
_C = 4096
_N = _B * _C
_PER_W = _N // _NW
_CHUNKS = 4
_CHUNK = _PER_W // _CHUNKS


def _gather_body(x_hbm, idx_hbm, out_hbm, idx_v, fidx_v, out_v, sem):
    wid = lax.axis_index("s") * _NC + lax.axis_index("c")
    e0 = wid * _PER_W
    b = e0 // _C
    cbase = e0 % _C

    pltpu.sync_copy(idx_hbm.at[b, 0, pl.ds(cbase, _PER_W)], idx_v)

    lane = lax.iota(jnp.int32, _L)
    for i in range(_PER_W // _L):
        c = cbase + i * _L
        v = idx_v[pl.ds(i * _L, _L)]
        r = v + b * _R
        addr = (
            (r >> 3) * 32768
            + (r & 7) * 128
            + ((c >> 7) * 1024 + (c & 127))
            + lane
        )
        fidx_v[pl.ds(i * _L, _L)] = addr

    copies = []
    for j in range(_CHUNKS):
        copies.append(
            pltpu.async_copy(
                x_hbm.at[fidx_v.at[pl.ds(j * _CHUNK, _CHUNK)]],
                out_v.at[pl.ds(j * _CHUNK, _CHUNK)],
                sem,
            )
        )
    for cp in copies:
        cp.wait()

    pltpu.sync_copy(out_v, out_hbm.at[b, 0, pl.ds(cbase, _PER_W)])


@jax.jit
def kernel(x, index):
    x_phys = (
        x.reshape(_B * _R // 8, 8, _C // 128, 128)
        .transpose(0, 2, 1, 3)
        .reshape(-1)
    )

    mesh = plsc.VectorSubcoreMesh(core_axis_name="c", subcore_axis_name="s")
    run = functools.partial(
        pl.kernel,
        mesh=mesh,
        out_type=jax.ShapeDtypeStruct((_B, 1, _C), jnp.float32),
        scratch_types=[
            pltpu.VMEM((_PER_W,), jnp.int32),
            pltpu.VMEM((_PER_W,), jnp.int32),
            pltpu.VMEM((_PER_W,), jnp.float32),
            pltpu.SemaphoreType.DMA,
        ],
    )(_gather_body)
    return run(x_phys, index)
